# baseline (speedup 1.0000x reference)
import functools

import jax
import jax.numpy as jnp
from jax import lax
from jax.experimental import pallas as pl
from jax.experimental.pallas import tpu as pltpu
from jax.experimental.pallas import tpu_sc as plsc


def _head_body(nsub, nk, bt, *refs):
    x_ref = refs[0]
    e_refs = refs[1 : 1 + nsub]
    w_ref = refs[1 + nsub]
    idx_ref = refs[2 + nsub]
    xres_ref = refs[3 + nsub]
    sx_ref = refs[4 + nsub]
    best_ref = refs[5 + nsub]
    bidx_ref = refs[6 + nsub]

    k = pl.program_id(0)
    i = pl.program_id(1)
    kt = w_ref.shape[0]
    bsl = pl.ds(i * bt, bt)

    @pl.when(k == 0)
    def _prep():
        x = x_ref[...]
        for e_ref in e_refs:
            x = x - e_ref[...].astype(jnp.bfloat16).astype(jnp.float32)
        xres_ref[bsl, :] = x
        sx_ref[0, bsl] = jnp.sum(x * x, axis=1)

    w = w_ref[...]
    wsq = jnp.sum(w * w, axis=1)
    mm2 = lax.dot_general(
        (-2.0 * w).astype(jnp.bfloat16),
        xres_ref[bsl, :].astype(jnp.bfloat16),
        dimension_numbers=(((1,), (1,)), ((), ())),
        preferred_element_type=jnp.float32,
    )
    t = (wsq[:, None] + sx_ref[0, bsl][None, :]) + mm2

    loc_min = jnp.min(t, axis=0)
    loc_arg = jnp.argmin(t, axis=0).astype(jnp.int32) + k * kt

    @pl.when(k == 0)
    def _init():
        best_ref[0, bsl] = loc_min
        bidx_ref[0, bsl] = loc_arg

    @pl.when(k > 0)
    def _update():
        prev = best_ref[0, bsl]
        upd = loc_min < prev
        best_ref[0, bsl] = jnp.where(upd, loc_min, prev)
        bidx_ref[0, bsl] = jnp.where(upd, loc_arg, bidx_ref[0, bsl])

    @pl.when(k == nk - 1)
    def _emit():
        idx_ref[0, 0, :] = bidx_ref[0, bsl]


def _head_argmin(w, x, es, bt=256, kt=8192):
    b, d = x.shape
    kk = w.shape[0]
    nb = b // bt
    nk = kk // kt
    nsub = len(es)

    first_sweep = lambda k, i: (jnp.where(k == 0, i, 0), 0)
    in_specs = [pl.BlockSpec((bt, d), first_sweep)]
    for _ in range(nsub):
        in_specs.append(pl.BlockSpec((bt, d), first_sweep))
    in_specs.append(pl.BlockSpec((kt, d), lambda k, i: (k, 0)))

    out = pl.pallas_call(
        functools.partial(_head_body, nsub, nk, bt),
        grid=(nk, nb),
        in_specs=in_specs,
        out_specs=pl.BlockSpec((1, 1, bt), lambda k, i: (i, 0, 0)),
        out_shape=jax.ShapeDtypeStruct((nb, 1, bt), jnp.int32),
        scratch_shapes=[
            pltpu.VMEM((b, d), jnp.float32),
            pltpu.VMEM((1, b), jnp.float32),
            pltpu.VMEM((1, b), jnp.float32),
            pltpu.VMEM((1, b), jnp.int32),
        ],
        compiler_params=pltpu.CompilerParams(
            dimension_semantics=("arbitrary", "arbitrary"),
        ),
    )(x, *es, w)
    return out.reshape(b)


def _sc_gather(table, idx):
    kk, d = table.shape
    b = idx.shape[0]
    info = plsc.get_sparse_core_info()
    nw = info.num_cores * info.num_subcores
    b_per_w = b // nw
    mesh = plsc.VectorSubcoreMesh(core_axis_name="c", subcore_axis_name="s")

    @functools.partial(
        pl.kernel,
        mesh=mesh,
        out_type=jax.ShapeDtypeStruct((b, d), jnp.float32),
        scratch_types=[
            pltpu.VMEM((b_per_w,), jnp.int32),
            pltpu.VMEM((b_per_w, d), jnp.float32),
            pltpu.SemaphoreType.DMA,
        ],
    )
    def gather(table_hbm, idx_hbm, out_hbm, idx_v, rows_v, sem):
        wid = lax.axis_index("s") * info.num_cores + lax.axis_index("c")
        base = wid * b_per_w
        pltpu.sync_copy(idx_hbm.at[pl.ds(base, b_per_w)], idx_v)
        pltpu.async_copy(table_hbm.at[idx_v], rows_v, sem).wait()
        pltpu.sync_copy(rows_v, out_hbm.at[pl.ds(base, b_per_w)])

    return gather(table, idx)


def _combine_body(e0_ref, e1_ref, e2_ref, out_ref):
    out_ref[...] = e0_ref[...] + e1_ref[...] + e2_ref[...]


def _combine(e0, e1, e2, bt=512):
    b, d = e0.shape
    nb = b // bt
    spec = pl.BlockSpec((bt, d), lambda i: (i, 0))
    return pl.pallas_call(
        _combine_body,
        grid=(nb,),
        in_specs=[spec, spec, spec],
        out_specs=spec,
        out_shape=jax.ShapeDtypeStruct((b, d), jnp.float32),
    )(e0, e1, e2)


def kernel(inputs, emb0, emb1, emb2):
    b = inputs.shape[0]
    x0 = inputs[:, 0, :]
    embs = (emb0, emb1, emb2)

    idxs = []
    es = []
    for h in range(3):
        idx = _head_argmin(embs[h], x0, es[:h])
        idxs.append(idx)
        es.append(_sc_gather(embs[h], idx))

    quantized = _combine(*es).reshape(b, 1, inputs.shape[2])
    codes = jnp.stack(idxs, axis=1)
    return quantized, codes


# speedup vs baseline: 1.3344x; 1.3344x over previous
import functools

import jax
import jax.numpy as jnp
from jax import lax
from jax.experimental import pallas as pl
from jax.experimental.pallas import tpu as pltpu
from jax.experimental.pallas import tpu_sc as plsc


def _head_body(nsub, nk, bt, *refs):
    x_ref = refs[0]
    e_refs = refs[1 : 1 + nsub]
    w_ref = refs[1 + nsub]
    idx_ref = refs[2 + nsub]
    xres_ref = refs[3 + nsub]
    sx_ref = refs[4 + nsub]
    best_ref = refs[5 + nsub]
    bidx_ref = refs[6 + nsub]
    wbf_ref = refs[7 + nsub]

    k = pl.program_id(0)
    i = pl.program_id(1)
    kt = w_ref.shape[0]
    bsl = pl.ds(i * bt, bt)

    @pl.when(k == 0)
    def _prep():
        x = x_ref[...]
        for e_ref in e_refs:
            x = x - e_ref[...].astype(jnp.bfloat16).astype(jnp.float32)
        xres_ref[bsl, :] = x
        sx_ref[0, bsl] = jnp.sum(x * x, axis=1)

    @pl.when(i == 0)
    def _wprep():
        wbf_ref[...] = (-2.0 * w_ref[...]).astype(jnp.bfloat16)

    w = w_ref[...]
    wsq = jnp.sum(w * w, axis=1)
    mm2 = lax.dot_general(
        wbf_ref[...],
        xres_ref[bsl, :].astype(jnp.bfloat16),
        dimension_numbers=(((1,), (1,)), ((), ())),
        preferred_element_type=jnp.float32,
    )
    t = (wsq[:, None] + sx_ref[0, bsl][None, :]) + mm2

    loc_min = jnp.min(t, axis=0)
    loc_arg = jnp.argmin(t, axis=0).astype(jnp.int32) + k * kt

    @pl.when(k == 0)
    def _init():
        best_ref[0, bsl] = loc_min
        bidx_ref[0, bsl] = loc_arg

    @pl.when(k > 0)
    def _update():
        prev = best_ref[0, bsl]
        upd = loc_min < prev
        best_ref[0, bsl] = jnp.where(upd, loc_min, prev)
        bidx_ref[0, bsl] = jnp.where(upd, loc_arg, bidx_ref[0, bsl])

    @pl.when(k == nk - 1)
    def _emit():
        idx_ref[0, 0, :] = bidx_ref[0, bsl]


def _head_argmin(w, x, es, bt=512, kt=8192):
    b, d = x.shape
    kk = w.shape[0]
    nb = b // bt
    nk = kk // kt
    nsub = len(es)

    first_sweep = lambda k, i: (jnp.where(k == 0, i, 0), 0)
    in_specs = [pl.BlockSpec((bt, d), first_sweep)]
    for _ in range(nsub):
        in_specs.append(pl.BlockSpec((bt, d), first_sweep))
    in_specs.append(pl.BlockSpec((kt, d), lambda k, i: (k, 0)))

    out = pl.pallas_call(
        functools.partial(_head_body, nsub, nk, bt),
        grid=(nk, nb),
        in_specs=in_specs,
        out_specs=pl.BlockSpec((1, 1, bt), lambda k, i: (i, 0, 0)),
        out_shape=jax.ShapeDtypeStruct((nb, 1, bt), jnp.int32),
        scratch_shapes=[
            pltpu.VMEM((b, d), jnp.float32),
            pltpu.VMEM((1, b), jnp.float32),
            pltpu.VMEM((1, b), jnp.float32),
            pltpu.VMEM((1, b), jnp.int32),
            pltpu.VMEM((kt, d), jnp.bfloat16),
        ],
        compiler_params=pltpu.CompilerParams(
            dimension_semantics=("arbitrary", "arbitrary"),
        ),
    )(x, *es, w)
    return out.reshape(b)


def _sc_gather(table, idx):
    kk, d = table.shape
    b = idx.shape[0]
    info = plsc.get_sparse_core_info()
    nw = info.num_cores * info.num_subcores
    b_per_w = b // nw
    mesh = plsc.VectorSubcoreMesh(core_axis_name="c", subcore_axis_name="s")

    @functools.partial(
        pl.kernel,
        mesh=mesh,
        out_type=jax.ShapeDtypeStruct((b, d), jnp.float32),
        scratch_types=[
            pltpu.VMEM((b_per_w,), jnp.int32),
            pltpu.VMEM((b_per_w, d), jnp.float32),
            pltpu.SemaphoreType.DMA,
        ],
    )
    def gather(table_hbm, idx_hbm, out_hbm, idx_v, rows_v, sem):
        wid = lax.axis_index("s") * info.num_cores + lax.axis_index("c")
        base = wid * b_per_w
        pltpu.sync_copy(idx_hbm.at[pl.ds(base, b_per_w)], idx_v)
        pltpu.async_copy(table_hbm.at[idx_v], rows_v, sem).wait()
        pltpu.sync_copy(rows_v, out_hbm.at[pl.ds(base, b_per_w)])

    return gather(table, idx)


def _combine_body(e0_ref, e1_ref, e2_ref, out_ref):
    out_ref[...] = e0_ref[...] + e1_ref[...] + e2_ref[...]


def _combine(e0, e1, e2, bt=512):
    b, d = e0.shape
    nb = b // bt
    spec = pl.BlockSpec((bt, d), lambda i: (i, 0))
    return pl.pallas_call(
        _combine_body,
        grid=(nb,),
        in_specs=[spec, spec, spec],
        out_specs=spec,
        out_shape=jax.ShapeDtypeStruct((b, d), jnp.float32),
    )(e0, e1, e2)


def kernel(inputs, emb0, emb1, emb2):
    b = inputs.shape[0]
    x0 = inputs[:, 0, :]
    embs = (emb0, emb1, emb2)

    idxs = []
    es = []
    for h in range(3):
        idx = _head_argmin(embs[h], x0, es[:h])
        idxs.append(idx)
        es.append(_sc_gather(embs[h], idx))

    quantized = _combine(*es).reshape(b, 1, inputs.shape[2])
    codes = jnp.stack(idxs, axis=1)
    return quantized, codes


# speedup vs baseline: 1.3636x; 1.0219x over previous
import functools

import jax
import jax.numpy as jnp
from jax import lax
from jax.experimental import pallas as pl
from jax.experimental.pallas import tpu as pltpu
from jax.experimental.pallas import tpu_sc as plsc


def _head_body(nsub, nk, bt, *refs):
    x_ref = refs[0]
    e_refs = refs[1 : 1 + nsub]
    w_ref = refs[1 + nsub]
    idx_ref = refs[2 + nsub]
    xres_ref = refs[3 + nsub]
    sx_ref = refs[4 + nsub]
    best_ref = refs[5 + nsub]
    bidx_ref = refs[6 + nsub]
    wbf_ref = refs[7 + nsub]
    wsq_ref = refs[8 + nsub]

    k = pl.program_id(0)
    i = pl.program_id(1)
    kt = w_ref.shape[0]
    bsl = pl.ds(i * bt, bt)

    @pl.when(k == 0)
    def _prep():
        x = x_ref[...]
        for e_ref in e_refs:
            x = x - e_ref[...].astype(jnp.bfloat16).astype(jnp.float32)
        xres_ref[bsl, :] = x
        sx_ref[0, bsl] = jnp.sum(x * x, axis=1)

    @pl.when(i == 0)
    def _wprep():
        w = w_ref[...]
        wbf_ref[...] = (-2.0 * w).astype(jnp.bfloat16)
        wsq_ref[...] = jnp.broadcast_to(
            jnp.sum(w * w, axis=1)[:, None], w.shape[:1] + (wsq_ref.shape[1],)
        )

    mm2 = lax.dot_general(
        wbf_ref[...],
        xres_ref[bsl, :].astype(jnp.bfloat16),
        dimension_numbers=(((1,), (1,)), ((), ())),
        preferred_element_type=jnp.float32,
    )
    t = (wsq_ref[...] + sx_ref[0, bsl][None, :]) + mm2

    loc_min = jnp.min(t, axis=0)
    loc_arg = jnp.argmin(t, axis=0).astype(jnp.int32) + k * kt

    @pl.when(k == 0)
    def _init():
        best_ref[0, bsl] = loc_min
        bidx_ref[0, bsl] = loc_arg

    @pl.when(k > 0)
    def _update():
        prev = best_ref[0, bsl]
        upd = loc_min < prev
        best_ref[0, bsl] = jnp.where(upd, loc_min, prev)
        bidx_ref[0, bsl] = jnp.where(upd, loc_arg, bidx_ref[0, bsl])

    @pl.when(k == nk - 1)
    def _emit():
        idx_ref[0, 0, :] = bidx_ref[0, bsl]


def _head_argmin(w, x, es, bt=512, kt=8192):
    b, d = x.shape
    kk = w.shape[0]
    nb = b // bt
    nk = kk // kt
    nsub = len(es)

    first_sweep = lambda k, i: (jnp.where(k == 0, i, 0), 0)
    in_specs = [pl.BlockSpec((bt, d), first_sweep)]
    for _ in range(nsub):
        in_specs.append(pl.BlockSpec((bt, d), first_sweep))
    in_specs.append(pl.BlockSpec((kt, d), lambda k, i: (k, 0)))

    out = pl.pallas_call(
        functools.partial(_head_body, nsub, nk, bt),
        grid=(nk, nb),
        in_specs=in_specs,
        out_specs=pl.BlockSpec((1, 1, bt), lambda k, i: (i, 0, 0)),
        out_shape=jax.ShapeDtypeStruct((nb, 1, bt), jnp.int32),
        scratch_shapes=[
            pltpu.VMEM((b, d), jnp.float32),
            pltpu.VMEM((1, b), jnp.float32),
            pltpu.VMEM((1, b), jnp.float32),
            pltpu.VMEM((1, b), jnp.int32),
            pltpu.VMEM((kt, d), jnp.bfloat16),
            pltpu.VMEM((kt, bt), jnp.float32),
        ],
        compiler_params=pltpu.CompilerParams(
            dimension_semantics=("arbitrary", "arbitrary"),
        ),
    )(x, *es, w)
    return out.reshape(b)


def _sc_gather(table, idx):
    kk, d = table.shape
    b = idx.shape[0]
    info = plsc.get_sparse_core_info()
    nw = info.num_cores * info.num_subcores
    b_per_w = b // nw
    mesh = plsc.VectorSubcoreMesh(core_axis_name="c", subcore_axis_name="s")

    @functools.partial(
        pl.kernel,
        mesh=mesh,
        out_type=jax.ShapeDtypeStruct((b, d), jnp.float32),
        scratch_types=[
            pltpu.VMEM((b_per_w,), jnp.int32),
            pltpu.VMEM((b_per_w, d), jnp.float32),
            pltpu.SemaphoreType.DMA,
        ],
    )
    def gather(table_hbm, idx_hbm, out_hbm, idx_v, rows_v, sem):
        wid = lax.axis_index("s") * info.num_cores + lax.axis_index("c")
        base = wid * b_per_w
        pltpu.sync_copy(idx_hbm.at[pl.ds(base, b_per_w)], idx_v)
        pltpu.async_copy(table_hbm.at[idx_v], rows_v, sem).wait()
        pltpu.sync_copy(rows_v, out_hbm.at[pl.ds(base, b_per_w)])

    return gather(table, idx)


def _combine_body(e0_ref, e1_ref, e2_ref, out_ref):
    out_ref[...] = e0_ref[...] + e1_ref[...] + e2_ref[...]


def _combine(e0, e1, e2, bt=512):
    b, d = e0.shape
    nb = b // bt
    spec = pl.BlockSpec((bt, d), lambda i: (i, 0))
    return pl.pallas_call(
        _combine_body,
        grid=(nb,),
        in_specs=[spec, spec, spec],
        out_specs=spec,
        out_shape=jax.ShapeDtypeStruct((b, d), jnp.float32),
    )(e0, e1, e2)


def kernel(inputs, emb0, emb1, emb2):
    b = inputs.shape[0]
    x0 = inputs[:, 0, :]
    embs = (emb0, emb1, emb2)

    idxs = []
    es = []
    for h in range(3):
        idx = _head_argmin(embs[h], x0, es[:h])
        idxs.append(idx)
        es.append(_sc_gather(embs[h], idx))

    quantized = _combine(*es).reshape(b, 1, inputs.shape[2])
    codes = jnp.stack(idxs, axis=1)
    return quantized, codes


# speedup vs baseline: 1.3726x; 1.0066x over previous
import functools

import jax
import jax.numpy as jnp
from jax import lax
from jax.experimental import pallas as pl
from jax.experimental.pallas import tpu as pltpu
from jax.experimental.pallas import tpu_sc as plsc


def _head_body(nsub, nk, bt, *refs):
    x_ref = refs[0]
    e_refs = refs[1 : 1 + nsub]
    w_ref = refs[1 + nsub]
    idx_ref = refs[2 + nsub]
    xres_ref = refs[3 + nsub]
    sx_ref = refs[4 + nsub]
    best_ref = refs[5 + nsub]
    bidx_ref = refs[6 + nsub]
    wbf_ref = refs[7 + nsub]
    wsq_ref = refs[8 + nsub]

    k = pl.program_id(0)
    i = pl.program_id(1)
    kt = w_ref.shape[0]
    bsl = pl.ds(i * bt, bt)

    @pl.when(k == 0)
    def _prep():
        x = x_ref[...]
        for e_ref in e_refs:
            x = x - e_ref[...].astype(jnp.bfloat16).astype(jnp.float32)
        xres_ref[bsl, :] = x
        sx_ref[0, bsl] = jnp.sum(x * x, axis=1)

    @pl.when(i == 0)
    def _wprep():
        w = w_ref[...]
        wbf_ref[...] = (-2.0 * w).astype(jnp.bfloat16)
        wsq_ref[...] = jnp.broadcast_to(
            jnp.sum(w * w, axis=1)[:, None], w.shape[:1] + (wsq_ref.shape[1],)
        )

    mm2 = lax.dot_general(
        wbf_ref[...],
        xres_ref[bsl, :].astype(jnp.bfloat16),
        dimension_numbers=(((1,), (1,)), ((), ())),
        preferred_element_type=jnp.float32,
    )
    t = (wsq_ref[...] + sx_ref[0, bsl][None, :]) + mm2

    if nk == 1:
        idx_ref[0, 0, :] = jnp.argmin(t, axis=0).astype(jnp.int32)
        return

    loc_min = jnp.min(t, axis=0)
    loc_arg = jnp.argmin(t, axis=0).astype(jnp.int32) + k * kt

    @pl.when(k == 0)
    def _init():
        best_ref[0, bsl] = loc_min
        bidx_ref[0, bsl] = loc_arg

    @pl.when(k > 0)
    def _update():
        prev = best_ref[0, bsl]
        upd = loc_min < prev
        best_ref[0, bsl] = jnp.where(upd, loc_min, prev)
        bidx_ref[0, bsl] = jnp.where(upd, loc_arg, bidx_ref[0, bsl])

    @pl.when(k == nk - 1)
    def _emit():
        idx_ref[0, 0, :] = bidx_ref[0, bsl]


def _head_argmin(w, x, es, bt=512, kt=8192):
    b, d = x.shape
    kk = w.shape[0]
    nb = b // bt
    nk = kk // kt
    nsub = len(es)

    first_sweep = lambda k, i: (jnp.where(k == 0, i, 0), 0)
    in_specs = [pl.BlockSpec((bt, d), first_sweep)]
    for _ in range(nsub):
        in_specs.append(pl.BlockSpec((bt, d), first_sweep))
    in_specs.append(pl.BlockSpec((kt, d), lambda k, i: (k, 0)))

    out = pl.pallas_call(
        functools.partial(_head_body, nsub, nk, bt),
        grid=(nk, nb),
        in_specs=in_specs,
        out_specs=pl.BlockSpec((1, 1, bt), lambda k, i: (i, 0, 0)),
        out_shape=jax.ShapeDtypeStruct((nb, 1, bt), jnp.int32),
        scratch_shapes=[
            pltpu.VMEM((b, d), jnp.float32),
            pltpu.VMEM((1, b), jnp.float32),
            pltpu.VMEM((1, b), jnp.float32),
            pltpu.VMEM((1, b), jnp.int32),
            pltpu.VMEM((kt, d), jnp.bfloat16),
            pltpu.VMEM((kt, bt), jnp.float32),
        ],
        compiler_params=pltpu.CompilerParams(
            dimension_semantics=("arbitrary", "arbitrary"),
        ),
    )(x, *es, w)
    return out.reshape(b)


def _sc_gather(table, idx):
    kk, d = table.shape
    b = idx.shape[0]
    info = plsc.get_sparse_core_info()
    nw = info.num_cores * info.num_subcores
    b_per_w = b // nw
    mesh = plsc.VectorSubcoreMesh(core_axis_name="c", subcore_axis_name="s")

    @functools.partial(
        pl.kernel,
        mesh=mesh,
        out_type=jax.ShapeDtypeStruct((b, d), jnp.float32),
        scratch_types=[
            pltpu.VMEM((b_per_w,), jnp.int32),
            pltpu.VMEM((b_per_w, d), jnp.float32),
            pltpu.SemaphoreType.DMA,
        ],
    )
    def gather(table_hbm, idx_hbm, out_hbm, idx_v, rows_v, sem):
        wid = lax.axis_index("s") * info.num_cores + lax.axis_index("c")
        base = wid * b_per_w
        pltpu.sync_copy(idx_hbm.at[pl.ds(base, b_per_w)], idx_v)
        pltpu.async_copy(table_hbm.at[idx_v], rows_v, sem).wait()
        pltpu.sync_copy(rows_v, out_hbm.at[pl.ds(base, b_per_w)])

    return gather(table, idx)


def _combine_body(e0_ref, e1_ref, e2_ref, out_ref):
    out_ref[...] = e0_ref[...] + e1_ref[...] + e2_ref[...]


def _combine(e0, e1, e2, bt=512):
    b, d = e0.shape
    nb = b // bt
    spec = pl.BlockSpec((bt, d), lambda i: (i, 0))
    return pl.pallas_call(
        _combine_body,
        grid=(nb,),
        in_specs=[spec, spec, spec],
        out_specs=spec,
        out_shape=jax.ShapeDtypeStruct((b, d), jnp.float32),
    )(e0, e1, e2)


def kernel(inputs, emb0, emb1, emb2):
    b = inputs.shape[0]
    x0 = inputs[:, 0, :]
    embs = (emb0, emb1, emb2)

    idxs = []
    es = []
    for h in range(3):
        idx = _head_argmin(embs[h], x0, es[:h])
        idxs.append(idx)
        es.append(_sc_gather(embs[h], idx))

    quantized = _combine(*es).reshape(b, 1, inputs.shape[2])
    codes = jnp.stack(idxs, axis=1)
    return quantized, codes
